# baseline (speedup 1.0000x reference)
import functools

import jax
import jax.numpy as jnp
from jax import lax
from jax.experimental import pallas as pl
from jax.experimental.pallas import tpu as pltpu
from jax.experimental.pallas import tpu_sc as plsc

_K = 16
_DTILE = 16


def _dist_body(x_ref, y_ref, o_ref):
    mm = x_ref.shape[2]
    c_dim = x_ref.shape[3]
    n = y_ref.shape[3]

    def dist_tile(mt, carry):
        base = mt * _DTILE
        acc = jnp.zeros((_DTILE, n), jnp.float32)
        for c0 in range(0, c_dim, 4):
            chain = None
            for c in range(c0, min(c0 + 4, c_dim)):
                xa = x_ref[0, 0, pl.ds(base, _DTILE), c:c + 1]
                yb = jnp.reshape(y_ref[0, 0, c, :], (1, n))
                d = xa - yb
                s = d * d
                chain = s if chain is None else chain + s
            acc = acc + chain
        o_ref[0, 0, pl.ds(base, _DTILE), :] = jnp.where(
            acc == 0.0, 0.0, acc * lax.rsqrt(acc))
        return carry

    lax.fori_loop(0, mm // _DTILE, dist_tile, 0)


def _make_sc_topk(nrows, rowlen):
    info = plsc.get_sparse_core_info()
    nc, ns = info.num_cores, info.num_subcores
    nw = nc * ns
    rpw = nrows // nw
    nquads = rpw // 4
    nchunks = rowlen // _K

    mesh = plsc.VectorSubcoreMesh(core_axis_name="c", subcore_axis_name="s")

    @functools.partial(
        pl.kernel, mesh=mesh,
        compiler_params=pltpu.CompilerParams(needs_layout_passes=False),
        out_type=[
            jax.ShapeDtypeStruct((nrows * _K,), jnp.float32),
            jax.ShapeDtypeStruct((nrows * _K,), jnp.int32),
        ],
        scratch_types=[
            pltpu.VMEM((8 * rowlen,), jnp.float32),
            pltpu.VMEM((rpw * _K,), jnp.float32),
            pltpu.VMEM((rpw * _K,), jnp.int32),
            pltpu.SemaphoreType.DMA,
            pltpu.SemaphoreType.DMA,
        ],
    )
    def sc_topk(mats_hbm, dk_hbm, di_hbm, rowbuf, okey, oidx, sem0, sem1):
        wid = lax.axis_index("s") * nc + lax.axis_index("c")
        gbase = wid * rpw
        iota = lax.iota(jnp.int32, 16)
        inf = jnp.float32(jnp.inf)

        def quad_dma(quad, slot, sem):
            src = mats_hbm.at[pl.ds((gbase + quad * 4) * rowlen, 4 * rowlen)]
            dst = rowbuf.at[pl.ds(slot * 4 * rowlen, 4 * rowlen)]
            return pltpu.make_async_copy(src, dst, sem)

        def process_rows(slotbase, rlocal):
            t0 = tuple(
                plsc.sort_key_val(rowbuf[pl.ds(slotbase + r * rowlen, _K)],
                                  iota)
                for r in range(4))

            def merge(t, ck, ci):
                tk, ti = t
                sk, si = plsc.sort_key_val(ck, ci)
                rk = lax.rev(tk, (0,))
                ri = lax.rev(ti, (0,))
                take = (sk < rk) | ((sk == rk) & (si < ri))
                nk = jnp.where(take, sk, rk)
                ni = jnp.where(take, si, ri)
                return plsc.sort_key_val(nk, ni)

            def chunk_step(c, ts):
                ci = iota + c * _K
                return tuple(
                    merge(ts[r],
                          rowbuf[pl.ds(slotbase + r * rowlen + c * _K, _K)],
                          ci)
                    for r in range(4))

            ts = lax.fori_loop(1, nchunks, chunk_step, t0)

            perm_odd = iota ^ 1
            perm_even = jnp.clip(((iota - 1) ^ 1) + 1, 0, _K - 1)

            def tie_pass(tk, ti, perm):
                pk = tk.at[perm].get(mode="promise_in_bounds")
                pi = ti.at[perm].get(mode="promise_in_bounds")
                eq = tk == pk
                lower = iota < perm
                return jnp.where(
                    eq,
                    jnp.where(lower, jnp.minimum(ti, pi),
                              jnp.maximum(ti, pi)),
                    ti)

            for r in range(4):
                tk, ti = ts[r]
                ti = tie_pass(tk, ti, perm_odd)
                ti = tie_pass(tk, ti, perm_even)
                ti = tie_pass(tk, ti, perm_odd)
                okey[pl.ds((rlocal + r) * _K, _K)] = tk
                oidx[pl.ds((rlocal + r) * _K, _K)] = ti

        quad_dma(0, 0, sem0).start()

        def outer(i, carry):
            q0 = i * 2
            quad_dma(q0, 0, sem0).wait()
            quad_dma(q0 + 1, 1, sem1).start()
            process_rows(0, q0 * 4)
            quad_dma(q0 + 1, 1, sem1).wait()

            @pl.when(q0 + 2 < nquads)
            def _():
                quad_dma(q0 + 2, 0, sem0).start()

            process_rows(4 * rowlen, q0 * 4 + 4)
            return carry

        lax.fori_loop(0, nquads // 2, outer, 0)
        pltpu.sync_copy(okey, dk_hbm.at[pl.ds(gbase * _K, rpw * _K)])
        pltpu.sync_copy(oidx, di_hbm.at[pl.ds(gbase * _K, rpw * _K)])

    return sc_topk


def kernel(a, b, k):
    del k
    bsz, c_dim, n = a.shape
    m = b.shape[2]

    xs = jnp.swapaxes(a, 1, 2)[None]
    ys = b[None]

    mats1 = pl.pallas_call(
        _dist_body,
        grid=(1, bsz),
        in_specs=[
            pl.BlockSpec((1, 1, m, c_dim), lambda d, bb: (d, bb, 0, 0)),
            pl.BlockSpec((1, 1, c_dim, n), lambda d, bb: (d, bb, 0, 0)),
        ],
        out_specs=pl.BlockSpec((1, 1, m, n), lambda d, bb: (d, bb, 0, 0)),
        out_shape=jax.ShapeDtypeStruct((1, bsz, m, n), jnp.float32),
    )(xs, ys)[0]

    mats0 = jnp.swapaxes(mats1, 1, 2)
    mats = jnp.stack([mats0, mats1])

    nrows = 2 * bsz * m
    dk, di = _make_sc_topk(nrows, n)(jnp.reshape(mats, (nrows * n,)))

    dk = jnp.reshape(dk, (2, bsz, m, _K))
    di = jnp.reshape(di, (2, bsz, m, _K))
    dist1p = jnp.transpose(dk[1], (0, 2, 1))
    dist2 = jnp.transpose(dk[0], (0, 2, 1))
    idx1p = jnp.transpose(di[1], (0, 2, 1)).astype(jnp.int64)
    idx2 = jnp.transpose(di[0], (0, 2, 1)).astype(jnp.int64)
    return (dist1p, dist2, idx1p, idx2)


# speedup vs baseline: 1.0006x; 1.0006x over previous
import functools

import jax
import jax.numpy as jnp
from jax import lax
from jax.experimental import pallas as pl
from jax.experimental.pallas import tpu as pltpu
from jax.experimental.pallas import tpu_sc as plsc

_K = 16
_DTILE = 16


def _dist_body(x_ref, y_ref, o_ref):
    mm = x_ref.shape[2]
    c_dim = x_ref.shape[3]
    n = y_ref.shape[3]

    def dist_tile(mt, carry):
        base = mt * _DTILE
        acc = jnp.zeros((_DTILE, n), jnp.float32)
        for c0 in range(0, c_dim, 4):
            chain = None
            for c in range(c0, min(c0 + 4, c_dim)):
                xa = x_ref[0, 0, pl.ds(base, _DTILE), c:c + 1]
                yb = jnp.reshape(y_ref[0, 0, c, :], (1, n))
                d = xa - yb
                s = d * d
                chain = s if chain is None else chain + s
            acc = acc + chain
        o_ref[0, 0, pl.ds(base, _DTILE), :] = jnp.where(
            acc == 0.0, 0.0, acc * lax.rsqrt(acc))
        return carry

    lax.fori_loop(0, mm // _DTILE, dist_tile, 0)


def _make_sc_topk(nrows, rowlen):
    info = plsc.get_sparse_core_info()
    nc, ns = info.num_cores, info.num_subcores
    nw = nc * ns
    rpw = nrows // nw
    nquads = rpw // 4
    nchunks = rowlen // _K

    mesh = plsc.VectorSubcoreMesh(core_axis_name="c", subcore_axis_name="s")

    @functools.partial(
        pl.kernel, mesh=mesh,
        compiler_params=pltpu.CompilerParams(needs_layout_passes=False),
        out_type=[
            jax.ShapeDtypeStruct((nrows * _K,), jnp.float32),
            jax.ShapeDtypeStruct((nrows * _K,), jnp.int32),
        ],
        scratch_types=[
            pltpu.VMEM((8 * rowlen,), jnp.float32),
            pltpu.VMEM((rpw * _K,), jnp.float32),
            pltpu.VMEM((rpw * _K,), jnp.int32),
            pltpu.SemaphoreType.DMA,
            pltpu.SemaphoreType.DMA,
        ],
    )
    def sc_topk(mats_hbm, dk_hbm, di_hbm, rowbuf, okey, oidx, sem0, sem1):
        wid = lax.axis_index("s") * nc + lax.axis_index("c")
        gbase = wid * rpw
        iota = lax.iota(jnp.int32, 16)
        inf = jnp.float32(jnp.inf)

        def quad_dma(quad, slot, sem):
            src = mats_hbm.at[pl.ds((gbase + quad * 4) * rowlen, 4 * rowlen)]
            dst = rowbuf.at[pl.ds(slot * 4 * rowlen, 4 * rowlen)]
            return pltpu.make_async_copy(src, dst, sem)

        def process_rows(slotbase, rlocal):
            t0 = tuple(
                plsc.sort_key_val(rowbuf[pl.ds(slotbase + r * rowlen, _K)],
                                  iota)
                for r in range(4))

            def merge(t, ck, ci):
                tk, ti = t
                sk, si = plsc.sort_key_val(ck, ci)
                rk = lax.rev(tk, (0,))
                ri = lax.rev(ti, (0,))
                take = (sk < rk) | ((sk == rk) & (si < ri))
                nk = jnp.where(take, sk, rk)
                ni = jnp.where(take, si, ri)
                return plsc.sort_key_val(nk, ni)

            def chunk_step(c, ts):
                ci = iota + c * _K
                return tuple(
                    merge(ts[r],
                          rowbuf[pl.ds(slotbase + r * rowlen + c * _K, _K)],
                          ci)
                    for r in range(4))

            ts = lax.fori_loop(1, nchunks, chunk_step, t0)

            perm_odd = iota ^ 1
            perm_even = jnp.clip(((iota - 1) ^ 1) + 1, 0, _K - 1)

            def tie_pass(tk, ti, perm):
                pk = tk.at[perm].get(mode="promise_in_bounds")
                pi = ti.at[perm].get(mode="promise_in_bounds")
                eq = tk == pk
                lower = iota < perm
                return jnp.where(
                    eq,
                    jnp.where(lower, jnp.minimum(ti, pi),
                              jnp.maximum(ti, pi)),
                    ti)

            for r in range(4):
                tk, ti = ts[r]
                ti = tie_pass(tk, ti, perm_odd)
                ti = tie_pass(tk, ti, perm_even)
                ti = tie_pass(tk, ti, perm_odd)
                okey[pl.ds((rlocal + r) * _K, _K)] = tk
                oidx[pl.ds((rlocal + r) * _K, _K)] = ti

        quad_dma(0, 0, sem0).start()

        def outer(i, carry):
            q0 = i * 2
            quad_dma(q0, 0, sem0).wait()
            quad_dma(q0 + 1, 1, sem1).start()
            process_rows(0, q0 * 4)
            quad_dma(q0 + 1, 1, sem1).wait()

            @pl.when(q0 + 2 < nquads)
            def _():
                quad_dma(q0 + 2, 0, sem0).start()

            process_rows(4 * rowlen, q0 * 4 + 4)
            return carry

        lax.fori_loop(0, nquads // 2, outer, 0)
        pltpu.sync_copy(okey, dk_hbm.at[pl.ds(gbase * _K, rpw * _K)])
        pltpu.sync_copy(oidx, di_hbm.at[pl.ds(gbase * _K, rpw * _K)])

    return sc_topk


def kernel(a, b, k):
    del k
    bsz, c_dim, n = a.shape
    m = b.shape[2]

    xs = jnp.swapaxes(a, 1, 2)[None]
    ys = b[None]

    mats1 = pl.pallas_call(
        _dist_body,
        grid=(1, bsz),
        in_specs=[
            pl.BlockSpec((1, 1, m, c_dim), lambda d, bb: (d, bb, 0, 0)),
            pl.BlockSpec((1, 1, c_dim, n), lambda d, bb: (d, bb, 0, 0)),
        ],
        out_specs=pl.BlockSpec((1, 1, m, n), lambda d, bb: (d, bb, 0, 0)),
        out_shape=jax.ShapeDtypeStruct((1, bsz, m, n), jnp.float32),
    )(xs, ys)[0]

    nrows = bsz * m
    sc_topk = _make_sc_topk(nrows, n)

    dk1, di1 = sc_topk(jnp.reshape(mats1, (nrows * n,)))
    mats0 = jnp.swapaxes(mats1, 1, 2)
    dk0, di0 = sc_topk(jnp.reshape(mats0, (nrows * n,)))

    dk1 = jnp.reshape(dk1, (bsz, n, _K))
    di1 = jnp.reshape(di1, (bsz, n, _K))
    dk0 = jnp.reshape(dk0, (bsz, m, _K))
    di0 = jnp.reshape(di0, (bsz, m, _K))
    dist1p = jnp.transpose(dk1, (0, 2, 1))
    dist2 = jnp.transpose(dk0, (0, 2, 1))
    idx1p = jnp.transpose(di1, (0, 2, 1)).astype(jnp.int64)
    idx2 = jnp.transpose(di0, (0, 2, 1)).astype(jnp.int64)
    return (dist1p, dist2, idx1p, idx2)


# speedup vs baseline: 1.0060x; 1.0054x over previous
import functools

import jax
import jax.numpy as jnp
from jax import lax
from jax.experimental import pallas as pl
from jax.experimental.pallas import tpu as pltpu
from jax.experimental.pallas import tpu_sc as plsc

_K = 16
_DTILE = 16


def _dist_body(x_ref, y_ref, o_ref):
    mm = x_ref.shape[2]
    c_dim = x_ref.shape[3]
    n = y_ref.shape[3]

    def dist_tile(mt, carry):
        base = mt * _DTILE
        acc = jnp.zeros((_DTILE, n), jnp.float32)
        for c0 in range(0, c_dim, 4):
            chain = None
            for c in range(c0, min(c0 + 4, c_dim)):
                xa = x_ref[0, 0, pl.ds(base, _DTILE), c:c + 1]
                yb = jnp.reshape(y_ref[0, 0, c, :], (1, n))
                d = xa - yb
                s = d * d
                chain = s if chain is None else chain + s
            acc = acc + chain
        o_ref[0, 0, pl.ds(base, _DTILE), :] = jnp.where(
            acc == 0.0, 0.0, acc * lax.rsqrt(acc))
        return carry

    lax.fori_loop(0, mm // _DTILE, dist_tile, 0)


def _make_sc_topk(nrows, rowlen):
    info = plsc.get_sparse_core_info()
    nc, ns = info.num_cores, info.num_subcores
    nw = nc * ns
    rpw = nrows // nw
    nquads = rpw // 4
    nchunks = rowlen // _K

    mesh = plsc.VectorSubcoreMesh(core_axis_name="c", subcore_axis_name="s")

    @functools.partial(
        pl.kernel, mesh=mesh,
        compiler_params=pltpu.CompilerParams(needs_layout_passes=False),
        out_type=[
            jax.ShapeDtypeStruct((nrows * _K,), jnp.float32),
            jax.ShapeDtypeStruct((nrows * _K,), jnp.int32),
            jax.ShapeDtypeStruct((nrows * _K,), jnp.float32),
            jax.ShapeDtypeStruct((nrows * _K,), jnp.int32),
        ],
        scratch_types=[
            pltpu.VMEM((8 * rowlen,), jnp.float32),
            pltpu.VMEM((rpw * _K,), jnp.float32),
            pltpu.VMEM((rpw * _K,), jnp.int32),
            pltpu.SemaphoreType.DMA,
            pltpu.SemaphoreType.DMA,
        ],
    )
    def sc_topk(m1_hbm, m0_hbm, dk1_hbm, di1_hbm, dk0_hbm, di0_hbm,
                rowbuf, okey, oidx, sem0, sem1):
        wid = lax.axis_index("s") * nc + lax.axis_index("c")
        gbase = wid * rpw
        iota = lax.iota(jnp.int32, 16)

        def quad_dma_on(mats_hbm, quad, slot, sem):
            src = mats_hbm.at[pl.ds((gbase + quad * 4) * rowlen, 4 * rowlen)]
            dst = rowbuf.at[pl.ds(slot * 4 * rowlen, 4 * rowlen)]
            return pltpu.make_async_copy(src, dst, sem)

        def process_rows(slotbase, rlocal):
            t0 = tuple(
                plsc.sort_key_val(rowbuf[pl.ds(slotbase + r * rowlen, _K)],
                                  iota)
                for r in range(4))

            def merge(t, ck, ci):
                tk, ti = t
                sk, si = plsc.sort_key_val(ck, ci)
                rk = lax.rev(tk, (0,))
                ri = lax.rev(ti, (0,))
                take = (sk < rk) | ((sk == rk) & (si < ri))
                nk = jnp.where(take, sk, rk)
                ni = jnp.where(take, si, ri)
                return plsc.sort_key_val(nk, ni)

            def chunk_step(c, ts):
                ci = iota + c * _K
                return tuple(
                    merge(ts[r],
                          rowbuf[pl.ds(slotbase + r * rowlen + c * _K, _K)],
                          ci)
                    for r in range(4))

            ts = lax.fori_loop(1, nchunks, chunk_step, t0)

            perm_odd = iota ^ 1
            perm_even = jnp.clip(((iota - 1) ^ 1) + 1, 0, _K - 1)

            def tie_pass(tk, ti, perm):
                pk = tk.at[perm].get(mode="promise_in_bounds")
                pi = ti.at[perm].get(mode="promise_in_bounds")
                eq = tk == pk
                lower = iota < perm
                return jnp.where(
                    eq,
                    jnp.where(lower, jnp.minimum(ti, pi),
                              jnp.maximum(ti, pi)),
                    ti)

            for r in range(4):
                tk, ti = ts[r]
                ti = tie_pass(tk, ti, perm_odd)
                ti = tie_pass(tk, ti, perm_even)
                ti = tie_pass(tk, ti, perm_odd)
                okey[pl.ds((rlocal + r) * _K, _K)] = tk
                oidx[pl.ds((rlocal + r) * _K, _K)] = ti

        for mats_hbm, dk_hbm, di_hbm in (
                (m1_hbm, dk1_hbm, di1_hbm), (m0_hbm, dk0_hbm, di0_hbm)):
            def quad_dma(quad, slot, sem, mh=mats_hbm):
                return quad_dma_on(mh, quad, slot, sem)

            quad_dma(0, 0, sem0).start()

            def outer(i, carry, quad_dma=quad_dma):
                q0 = i * 2
                quad_dma(q0, 0, sem0).wait()
                quad_dma(q0 + 1, 1, sem1).start()
                process_rows(0, q0 * 4)
                quad_dma(q0 + 1, 1, sem1).wait()

                @pl.when(q0 + 2 < nquads)
                def _():
                    quad_dma(q0 + 2, 0, sem0).start()

                process_rows(4 * rowlen, q0 * 4 + 4)
                return carry

            lax.fori_loop(0, nquads // 2, outer, 0)
            pltpu.sync_copy(okey, dk_hbm.at[pl.ds(gbase * _K, rpw * _K)])
            pltpu.sync_copy(oidx, di_hbm.at[pl.ds(gbase * _K, rpw * _K)])

    return sc_topk


def kernel(a, b, k):
    del k
    bsz, c_dim, n = a.shape
    m = b.shape[2]

    xs = jnp.swapaxes(a, 1, 2)[None]
    ys = b[None]

    mats1 = pl.pallas_call(
        _dist_body,
        grid=(1, bsz),
        in_specs=[
            pl.BlockSpec((1, 1, m, c_dim), lambda d, bb: (d, bb, 0, 0)),
            pl.BlockSpec((1, 1, c_dim, n), lambda d, bb: (d, bb, 0, 0)),
        ],
        out_specs=pl.BlockSpec((1, 1, m, n), lambda d, bb: (d, bb, 0, 0)),
        out_shape=jax.ShapeDtypeStruct((1, bsz, m, n), jnp.float32),
    )(xs, ys)[0]

    nrows = bsz * m
    mats0 = jnp.swapaxes(mats1, 1, 2)
    dk1, di1, dk0, di0 = _make_sc_topk(nrows, n)(
        jnp.reshape(mats1, (nrows * n,)), jnp.reshape(mats0, (nrows * n,)))

    dk1 = jnp.reshape(dk1, (bsz, n, _K))
    di1 = jnp.reshape(di1, (bsz, n, _K))
    dk0 = jnp.reshape(dk0, (bsz, m, _K))
    di0 = jnp.reshape(di0, (bsz, m, _K))
    dist1p = jnp.transpose(dk1, (0, 2, 1))
    dist2 = jnp.transpose(dk0, (0, 2, 1))
    idx1p = jnp.transpose(di1, (0, 2, 1)).astype(jnp.int64)
    idx2 = jnp.transpose(di0, (0, 2, 1)).astype(jnp.int64)
    return (dist1p, dist2, idx1p, idx2)
